# baseline (speedup 1.0000x reference)
import functools

import jax
import jax.numpy as jnp
from jax import lax
from jax.experimental import pallas as pl
from jax.experimental.pallas import tpu as pltpu
from jax.experimental.pallas import tpu_sc as plsc

BATCH = 4096
SEQ = 200
DIM = 64
TOTAL = BATCH * SEQ
NUM_WORKERS = 32
PER_WORKER = TOTAL // NUM_WORKERS
CHUNK = 128
FIRE = 8
TILE = CHUNK * FIRE
N_OUTER = PER_WORKER // TILE
N_CHUNKS = PER_WORKER // CHUNK


def _make_gather():
    mesh = plsc.VectorSubcoreMesh(core_axis_name="c", subcore_axis_name="s")

    @functools.partial(
        pl.kernel,
        mesh=mesh,
        out_type=jax.ShapeDtypeStruct((TOTAL, DIM), jnp.float32),
        scratch_types=[
            pltpu.VMEM((N_CHUNKS, CHUNK), jnp.int32),
            pltpu.VMEM((TILE, DIM), jnp.float32),
            pltpu.SemaphoreType.DMA,
        ],
        compiler_params=pltpu.CompilerParams(use_tc_tiling_on_sc=False),
    )
    def gather_kernel(idx_hbm, table_hbm, out_hbm, idx_v, rows_v, sem):
        wid = lax.axis_index("s") * 2 + lax.axis_index("c")
        base = wid * PER_WORKER
        pltpu.sync_copy(idx_hbm.at[wid], idx_v)

        def outer(g, carry):
            copies = []
            for b in range(FIRE):
                copies.append(
                    pltpu.async_copy(
                        table_hbm.at[idx_v.at[g * FIRE + b]],
                        rows_v.at[pl.ds(b * CHUNK, CHUNK)],
                        sem,
                    )
                )
            for c in copies:
                c.wait()
            pltpu.sync_copy(rows_v, out_hbm.at[pl.ds(base + g * TILE, TILE)])
            return carry

        lax.fori_loop(0, N_OUTER, outer, 0)

    return gather_kernel


_gather = _make_gather()


def kernel(word_inputs, word_seq_lengths, char_inputs, char_seq_lengths,
           char_seq_recover, word_embeddings):
    idx = word_inputs.astype(jnp.int32).reshape(NUM_WORKERS, N_CHUNKS, CHUNK)
    out = _gather(idx, word_embeddings)
    return out.reshape(BATCH, SEQ, DIM)


# speedup vs baseline: 1.0050x; 1.0050x over previous
import functools

import jax
import jax.numpy as jnp
from jax import lax
from jax.experimental import pallas as pl
from jax.experimental.pallas import tpu as pltpu
from jax.experimental.pallas import tpu_sc as plsc

BATCH = 4096
SEQ = 200
DIM = 64
TOTAL = BATCH * SEQ
NUM_WORKERS = 32
PER_WORKER = TOTAL // NUM_WORKERS
CHUNK = 128
FIRE = 4
TILE = CHUNK * FIRE
N_TILES = PER_WORKER // TILE
N_PAIRS = N_TILES // 2
N_CHUNKS = PER_WORKER // CHUNK


def _make_gather():
    mesh = plsc.VectorSubcoreMesh(core_axis_name="c", subcore_axis_name="s")

    @functools.partial(
        pl.kernel,
        mesh=mesh,
        out_type=jax.ShapeDtypeStruct((TOTAL, DIM), jnp.float32),
        scratch_types=[
            pltpu.VMEM((N_CHUNKS, CHUNK), jnp.int32),
            pltpu.VMEM((TILE, DIM), jnp.float32),
            pltpu.VMEM((TILE, DIM), jnp.float32),
            pltpu.SemaphoreType.DMA,
            pltpu.SemaphoreType.DMA,
            pltpu.SemaphoreType.DMA,
            pltpu.SemaphoreType.DMA,
        ],
        compiler_params=pltpu.CompilerParams(use_tc_tiling_on_sc=False),
    )
    def gather_kernel(idx_hbm, table_hbm, out_hbm, idx_v, rows0, rows1,
                      sg0, sg1, ss0, ss1):
        wid = lax.axis_index("s") * 2 + lax.axis_index("c")
        base = wid * PER_WORKER
        pltpu.sync_copy(idx_hbm.at[wid], idx_v)

        def fire(tile, buf, sem):
            return [
                pltpu.async_copy(
                    table_hbm.at[idx_v.at[tile * FIRE + b]],
                    buf.at[pl.ds(b * CHUNK, CHUNK)],
                    sem,
                )
                for b in range(FIRE)
            ]

        def scat_start(tile, buf, sem):
            pltpu.async_copy(buf, out_hbm.at[pl.ds(base + tile * TILE, TILE)],
                             sem)

        def scat_wait(buf, sem):
            pltpu.make_async_copy(buf, out_hbm.at[pl.ds(base, TILE)],
                                  sem).wait()

        def pair(p, first):
            t0 = 2 * p
            t1 = t0 + 1
            if not first:
                scat_wait(rows0, ss0)
            c0 = fire(t0, rows0, sg0)
            if not first:
                scat_wait(rows1, ss1)
            c1 = fire(t1, rows1, sg1)
            for c in c0:
                c.wait()
            scat_start(t0, rows0, ss0)
            for c in c1:
                c.wait()
            scat_start(t1, rows1, ss1)

        pair(0, True)

        def body(p, carry):
            pair(p, False)
            return carry

        lax.fori_loop(1, N_PAIRS, body, 0)
        scat_wait(rows0, ss0)
        scat_wait(rows1, ss1)

    return gather_kernel


_gather = _make_gather()


def kernel(word_inputs, word_seq_lengths, char_inputs, char_seq_lengths,
           char_seq_recover, word_embeddings):
    idx = word_inputs.astype(jnp.int32).reshape(NUM_WORKERS, N_CHUNKS, CHUNK)
    out = _gather(idx, word_embeddings)
    return out.reshape(BATCH, SEQ, DIM)
